# baseline (speedup 1.0000x reference)
import functools

import jax
import jax.numpy as jnp
from jax import lax
from jax.experimental import pallas as pl
from jax.experimental.pallas import tpu as pltpu
from jax.experimental.pallas import tpu_sc as plsc

BATCH = 16384
D = 32
STRIP = 128
NC = 2
NS = 16
NW = NC * NS
BPW = BATCH // NW
G = 16
NG = BPW // G
R = 4
NR = 8

_mesh = plsc.VectorSubcoreMesh(core_axis_name="c", subcore_axis_name="s")


@functools.partial(
    pl.kernel,
    out_type=jax.ShapeDtypeStruct((BATCH,), jnp.float32),
    mesh=_mesh,
    scratch_types=[
        pltpu.VMEM((BPW,), jnp.int32),
        pltpu.VMEM((BPW,), jnp.int32),
        pltpu.VMEM((2, R, D, STRIP), jnp.float32),
        pltpu.VMEM((G, D), jnp.float32),
        pltpu.VMEM((G, D), jnp.float32),
        pltpu.VMEM((48,), jnp.float32),
        pltpu.VMEM((BPW,), jnp.float32),
        pltpu.SemaphoreType.DMA,
    ],
    compiler_params=pltpu.CompilerParams(
        needs_layout_passes=False, use_tc_tiling_on_sc=True),
)
def _gmf_sc(uid_hbm, iid_hbm, ut_hbm, it_hbm, wb_hbm, out_hbm,
            uidx, iidx, strips, urows, irows, wv, outv, sem):
    wid = lax.axis_index("s") * NC + lax.axis_index("c")
    base = wid * BPW

    pltpu.sync_copy(uid_hbm.at[pl.ds(base, BPW)], uidx)
    pltpu.sync_copy(iid_hbm.at[pl.ds(base, BPW)], iidx)
    pltpu.sync_copy(wb_hbm, wv)

    lanes = lax.iota(jnp.int32, 16)
    d_lo = lanes
    d_hi = lanes + 16
    w_lo = wv[pl.ds(0, 16)]
    w_hi = wv[pl.ds(16, 16)]
    b0 = wv[pl.ds(32, 16)][0]

    def starts(off):
        u = uidx[pl.ds(off, 16)]
        i = iidx[pl.ds(off, 16)]
        return u & ~127, i & ~127

    def fire(r, slot, ustart, istart):
        tab = ut_hbm if r < R else it_hbm
        start = ustart if r < R else istart
        for j in range(R):
            s = pl.multiple_of(start[(r % R) * R + j], 128)
            pltpu.async_copy(tab.at[:, pl.ds(s, STRIP)],
                             strips.at[slot, j], sem)

    us0, is0 = starts(0)
    fire(0, 0, us0, is0)

    def group_body(g, _):
        off = pl.multiple_of(g * G, G)
        ustart, istart = starts(off)
        ulane = uidx[pl.ds(off, 16)] & 127
        ilane = iidx[pl.ds(off, 16)] & 127
        for r in range(NR):
            slot = r % 2
            for j in range(R):
                pltpu.make_async_copy(ut_hbm.at[:, pl.ds(0, STRIP)],
                                      strips.at[slot, j], sem).wait()
            if r + 1 < NR:
                fire(r + 1, 1 - slot, ustart, istart)
            else:
                @pl.when(g + 1 < NG)
                def _():
                    un, inx = starts(off + G)
                    fire(0, 1 - slot, un, inx)
            lane = ulane if r < R else ilane
            rows_ref = urows if r < R else irows
            for j in range(R):
                jj = (r % R) * R + j
                sv = jnp.full((16,), slot, jnp.int32)
                jv = jnp.full((16,), j, jnp.int32)
                wl = jnp.full((16,), lane[jj], jnp.int32)
                v_lo = plsc.load_gather(strips, [sv, jv, d_lo, wl])
                v_hi = plsc.load_gather(strips, [sv, jv, d_hi, wl])
                rows_ref[jj, pl.ds(0, 16)] = v_lo
                rows_ref[jj, pl.ds(16, 16)] = v_hi
        acc = jnp.zeros((16,), jnp.float32)
        for j in range(G):
            s = jnp.sum(urows[j, pl.ds(0, 16)] * irows[j, pl.ds(0, 16)] * w_lo
                        + urows[j, pl.ds(16, 16)] * irows[j, pl.ds(16, 16)] * w_hi)
            acc = jnp.where(lanes == j, s, acc)
        outv[pl.ds(off, 16)] = 1.0 / (1.0 + jnp.exp(-(acc + b0)))
        return 0

    lax.fori_loop(0, NG, group_body, 0)

    pltpu.sync_copy(outv, out_hbm.at[pl.ds(base, BPW)])


def kernel(user_ids, item_ids, user_table, item_table, W, b):
    wb = jnp.zeros((48,), jnp.float32)
    wb = wb.at[:D].set(W.reshape(D)).at[D].set(b[0])
    return _gmf_sc(user_ids.astype(jnp.int32), item_ids.astype(jnp.int32),
                   user_table.T, item_table.T, wb)


# speedup vs baseline: 1.1448x; 1.1448x over previous
import functools

import jax
import jax.numpy as jnp
from jax import lax
from jax.experimental import pallas as pl
from jax.experimental.pallas import tpu as pltpu
from jax.experimental.pallas import tpu_sc as plsc

BATCH = 16384
D = 32
STRIP = 128
NC = 2
NS = 16
NW = NC * NS
BPW = BATCH // NW
G = 16
NG = BPW // G
RB = 7

_ROUNDS = [(0, 0, 7), (0, 7, 7), (0, 14, 2),
           (1, 0, 7), (1, 7, 7), (1, 14, 2)]

_mesh = plsc.VectorSubcoreMesh(core_axis_name="c", subcore_axis_name="s")


@functools.partial(
    pl.kernel,
    out_type=jax.ShapeDtypeStruct((BATCH,), jnp.float32),
    mesh=_mesh,
    scratch_types=[
        pltpu.VMEM((BPW,), jnp.int32),
        pltpu.VMEM((BPW,), jnp.int32),
        pltpu.VMEM((2, RB, D, STRIP), jnp.float32),
        pltpu.VMEM((G, D), jnp.float32),
        pltpu.VMEM((G, D), jnp.float32),
        pltpu.VMEM((48,), jnp.float32),
        pltpu.VMEM((BPW,), jnp.float32),
        pltpu.SemaphoreType.DMA,
    ],
    compiler_params=pltpu.CompilerParams(
        needs_layout_passes=False, use_tc_tiling_on_sc=True),
)
def _gmf_sc(uid_hbm, iid_hbm, ut_hbm, it_hbm, wb_hbm, out_hbm,
            uidx, iidx, strips, urows, irows, wv, outv, sem):
    wid = lax.axis_index("s") * NC + lax.axis_index("c")
    base = wid * BPW

    pltpu.sync_copy(uid_hbm.at[pl.ds(base, BPW)], uidx)
    pltpu.sync_copy(iid_hbm.at[pl.ds(base, BPW)], iidx)
    pltpu.sync_copy(wb_hbm, wv)

    lanes = lax.iota(jnp.int32, 16)
    d_lo = lanes
    d_hi = lanes + 16
    w_lo = wv[pl.ds(0, 16)]
    w_hi = wv[pl.ds(16, 16)]
    b0 = wv[pl.ds(32, 16)][0]

    def fire(rnd, slot, ustart, istart):
        tab_sel, jbase, cnt = _ROUNDS[rnd]
        tab = ut_hbm if tab_sel == 0 else it_hbm
        start = ustart if tab_sel == 0 else istart
        for j in range(cnt):
            s = pl.multiple_of(start[jbase + j], 128)
            pltpu.async_copy(tab.at[:, pl.ds(s, STRIP)],
                             strips.at[slot, j], sem)

    def starts(off):
        return uidx[pl.ds(off, 16)] & ~127, iidx[pl.ds(off, 16)] & ~127

    us0, is0 = starts(0)
    fire(0, 0, us0, is0)

    def group_body(g, _):
        off = pl.multiple_of(g * G, G)
        ustart, istart = starts(off)
        ulane = uidx[pl.ds(off, 16)] & 127
        ilane = iidx[pl.ds(off, 16)] & 127
        for rnd, (tab_sel, jbase, cnt) in enumerate(_ROUNDS):
            slot = rnd % 2
            for j in range(cnt):
                pltpu.make_async_copy(ut_hbm.at[:, pl.ds(0, STRIP)],
                                      strips.at[slot, j], sem).wait()
            if rnd + 1 < len(_ROUNDS):
                fire(rnd + 1, 1 - slot, ustart, istart)
            else:
                @pl.when(g + 1 < NG)
                def _():
                    un, inx = starts(off + G)
                    fire(0, 1 - slot, un, inx)
            lane = ulane if tab_sel == 0 else ilane
            rows_ref = urows if tab_sel == 0 else irows
            for j in range(cnt):
                jj = jbase + j
                sv = jnp.full((16,), slot, jnp.int32)
                jv = jnp.full((16,), j, jnp.int32)
                wl = jnp.full((16,), lane[jj], jnp.int32)
                v_lo = plsc.load_gather(strips, [sv, jv, d_lo, wl])
                v_hi = plsc.load_gather(strips, [sv, jv, d_hi, wl])
                rows_ref[jj, pl.ds(0, 16)] = v_lo
                rows_ref[jj, pl.ds(16, 16)] = v_hi
        acc = jnp.zeros((16,), jnp.float32)
        for j in range(G):
            s = jnp.sum(urows[j, pl.ds(0, 16)] * irows[j, pl.ds(0, 16)] * w_lo
                        + urows[j, pl.ds(16, 16)] * irows[j, pl.ds(16, 16)] * w_hi)
            acc = jnp.where(lanes == j, s, acc)
        outv[pl.ds(off, 16)] = 1.0 / (1.0 + jnp.exp(-(acc + b0)))
        return 0

    lax.fori_loop(0, NG, group_body, 0)

    pltpu.sync_copy(outv, out_hbm.at[pl.ds(base, BPW)])


def kernel(user_ids, item_ids, user_table, item_table, W, b):
    wb = jnp.zeros((48,), jnp.float32)
    wb = wb.at[:D].set(W.reshape(D)).at[D].set(b[0])
    return _gmf_sc(user_ids.astype(jnp.int32), item_ids.astype(jnp.int32),
                   user_table.T, item_table.T, wb)


# speedup vs baseline: 1.2236x; 1.0688x over previous
import functools

import jax
import jax.numpy as jnp
from jax import lax
from jax.experimental import pallas as pl
from jax.experimental.pallas import tpu as pltpu
from jax.experimental.pallas import tpu_sc as plsc

BATCH = 16384
D = 32
STRIP = 128
NC = 2
NS = 16
NW = NC * NS
BPW = BATCH // NW
G = 16
NG = BPW // G
GF = 8

_mesh = plsc.VectorSubcoreMesh(core_axis_name="c", subcore_axis_name="s")


@functools.partial(
    pl.kernel,
    out_type=jax.ShapeDtypeStruct((BATCH,), jnp.float32),
    mesh=_mesh,
    scratch_types=[
        pltpu.VMEM((BPW,), jnp.int32),
        pltpu.VMEM((BPW,), jnp.int32),
        pltpu.VMEM((GF, D, STRIP), jnp.float32),
        pltpu.VMEM((G, D), jnp.float32),
        pltpu.VMEM((G, D), jnp.float32),
        pltpu.VMEM((48,), jnp.float32),
        pltpu.VMEM((BPW,), jnp.float32),
        pltpu.SemaphoreType.DMA,
    ],
    compiler_params=pltpu.CompilerParams(
        needs_layout_passes=False, use_tc_tiling_on_sc=True),
)
def _gmf_sc(uid_hbm, iid_hbm, ut_hbm, it_hbm, wb_hbm, out_hbm,
            uidx, iidx, strips, urows, irows, wv, outv, sem):
    wid = lax.axis_index("s") * NC + lax.axis_index("c")
    base = wid * BPW

    pltpu.sync_copy(uid_hbm.at[pl.ds(base, BPW)], uidx)
    pltpu.sync_copy(iid_hbm.at[pl.ds(base, BPW)], iidx)
    pltpu.sync_copy(wb_hbm, wv)

    lanes = lax.iota(jnp.int32, 16)
    d_lo = lanes
    d_hi = lanes + 16

    w_lo = wv[pl.ds(0, 16)]
    w_hi = wv[pl.ds(16, 16)]
    b0 = wv[pl.ds(32, 16)][0]

    def fetch_extract(tab_hbm, start, lane, rows_ref):
        for jbase in (0, GF):
            for j in range(GF):
                s = pl.multiple_of(start[jbase + j], 128)
                pltpu.async_copy(tab_hbm.at[:, pl.ds(s, STRIP)],
                                 strips.at[j], sem)
            for j in range(GF):
                pltpu.make_async_copy(tab_hbm.at[:, pl.ds(0, STRIP)],
                                      strips.at[j], sem).wait()
            for j in range(GF):
                jv = jnp.full((16,), j, jnp.int32)
                wl = jnp.full((16,), lane[jbase + j], jnp.int32)
                v_lo = plsc.load_gather(strips, [jv, d_lo, wl])
                v_hi = plsc.load_gather(strips, [jv, d_hi, wl])
                rows_ref[jbase + j, pl.ds(0, 16)] = v_lo
                rows_ref[jbase + j, pl.ds(16, 16)] = v_hi

    def group_body(g, _):
        off = pl.multiple_of(g * G, G)
        uids = uidx[pl.ds(off, 16)]
        iids = iidx[pl.ds(off, 16)]
        fetch_extract(ut_hbm, uids & ~127, uids & 127, urows)
        fetch_extract(it_hbm, iids & ~127, iids & 127, irows)
        acc = jnp.zeros((16,), jnp.float32)
        for j in range(G):
            s = jnp.sum(urows[j, pl.ds(0, 16)] * irows[j, pl.ds(0, 16)] * w_lo
                        + urows[j, pl.ds(16, 16)] * irows[j, pl.ds(16, 16)] * w_hi)
            acc = jnp.where(lanes == j, s, acc)
        outv[pl.ds(off, 16)] = 1.0 / (1.0 + jnp.exp(-(acc + b0)))
        return 0

    lax.fori_loop(0, NG, group_body, 0)

    pltpu.sync_copy(outv, out_hbm.at[pl.ds(base, BPW)])


def kernel(user_ids, item_ids, user_table, item_table, W, b):
    wb = jnp.zeros((48,), jnp.float32)
    wb = wb.at[:D].set(W.reshape(D)).at[D].set(b[0])
    return _gmf_sc(user_ids.astype(jnp.int32), item_ids.astype(jnp.int32),
                   user_table.T, item_table.T, wb)


# speedup vs baseline: 1.8510x; 1.5127x over previous
import functools

import jax
import jax.numpy as jnp
from jax import lax
from jax.experimental import pallas as pl
from jax.experimental.pallas import tpu as pltpu
from jax.experimental.pallas import tpu_sc as plsc

BATCH = 16384
D = 32
STRIP = 128
NC = 2
NS = 16
NW = NC * NS
BPW = BATCH // NW
G = 16
NG = BPW // G
NB = 8

_mesh = plsc.VectorSubcoreMesh(core_axis_name="c", subcore_axis_name="s")


@functools.partial(
    pl.kernel,
    out_type=jax.ShapeDtypeStruct((BATCH,), jnp.float32),
    mesh=_mesh,
    scratch_types=[
        pltpu.VMEM((BPW,), jnp.int32),
        pltpu.VMEM((BPW,), jnp.int32),
        pltpu.VMEM((NB, D, STRIP), jnp.float32),
        pltpu.VMEM((G, D), jnp.float32),
        pltpu.VMEM((G, D), jnp.float32),
        pltpu.VMEM((48,), jnp.float32),
        pltpu.VMEM((BPW,), jnp.float32),
        pltpu.SemaphoreType.DMA,
    ],
    compiler_params=pltpu.CompilerParams(
        needs_layout_passes=False, use_tc_tiling_on_sc=True),
)
def _gmf_sc(uid_hbm, iid_hbm, ut_hbm, it_hbm, wb_hbm, out_hbm,
            uidx, iidx, strips, urows, irows, wv, outv, sem):
    wid = lax.axis_index("s") * NC + lax.axis_index("c")
    base = wid * BPW

    pltpu.sync_copy(uid_hbm.at[pl.ds(base, BPW)], uidx)
    pltpu.sync_copy(iid_hbm.at[pl.ds(base, BPW)], iidx)
    pltpu.sync_copy(wb_hbm, wv)

    lanes = lax.iota(jnp.int32, 16)
    d_lo = lanes
    d_hi = lanes + 16

    w_lo = wv[pl.ds(0, 16)]
    w_hi = wv[pl.ds(16, 16)]
    b0 = wv[pl.ds(32, 16)][0]

    def fire_one(tab_hbm, start_vec, jj, slot):
        s = pl.multiple_of(start_vec[jj], 128)
        pltpu.async_copy(tab_hbm.at[:, pl.ds(s, STRIP)], strips.at[slot], sem)

    def group_starts(off):
        return uidx[pl.ds(off, 16)] & ~127, iidx[pl.ds(off, 16)] & ~127

    us0, _ = group_starts(0)
    for j in range(NB):
        fire_one(ut_hbm, us0, j, j)

    def group_body(g, _):
        off = pl.multiple_of(g * G, G)
        ustart, istart = group_starts(off)
        ulane = uidx[pl.ds(off, 16)] & 127
        ilane = iidx[pl.ds(off, 16)] & 127
        for fi in range(2 * G):
            slot = fi % NB
            pltpu.make_async_copy(ut_hbm.at[:, pl.ds(0, STRIP)],
                                  strips.at[slot], sem).wait()
            nxt = fi + NB
            if nxt < G:
                fire_one(ut_hbm, ustart, nxt, slot)
            elif nxt < 2 * G:
                fire_one(it_hbm, istart, nxt - G, slot)
            else:
                @pl.when(g + 1 < NG)
                def _():
                    un, _ = group_starts(off + G)
                    fire_one(ut_hbm, un, nxt - 2 * G, slot)
            jj = fi % G
            lane = ulane if fi < G else ilane
            rows_ref = urows if fi < G else irows
            sv = jnp.full((16,), slot, jnp.int32)
            wl = jnp.full((16,), lane[jj], jnp.int32)
            v_lo = plsc.load_gather(strips, [sv, d_lo, wl])
            v_hi = plsc.load_gather(strips, [sv, d_hi, wl])
            rows_ref[jj, pl.ds(0, 16)] = v_lo
            rows_ref[jj, pl.ds(16, 16)] = v_hi
        acc = jnp.zeros((16,), jnp.float32)
        for j in range(G):
            s = jnp.sum(urows[j, pl.ds(0, 16)] * irows[j, pl.ds(0, 16)] * w_lo
                        + urows[j, pl.ds(16, 16)] * irows[j, pl.ds(16, 16)] * w_hi)
            acc = jnp.where(lanes == j, s, acc)
        outv[pl.ds(off, 16)] = 1.0 / (1.0 + jnp.exp(-(acc + b0)))
        return 0

    lax.fori_loop(0, NG, group_body, 0)

    pltpu.sync_copy(outv, out_hbm.at[pl.ds(base, BPW)])


def kernel(user_ids, item_ids, user_table, item_table, W, b):
    wb = jnp.zeros((48,), jnp.float32)
    wb = wb.at[:D].set(W.reshape(D)).at[D].set(b[0])
    return _gmf_sc(user_ids.astype(jnp.int32), item_ids.astype(jnp.int32),
                   user_table.T, item_table.T, wb)
